# baseline (speedup 1.0000x reference)
import functools

import jax
import jax.numpy as jnp
from jax import lax
from jax.experimental import pallas as pl
from jax.experimental.pallas import tpu as pltpu
from jax.experimental.pallas import tpu_sc as plsc

_BLK = 2048


def _tc_stream_body(nsteps, U_real, x_ref, pu_ref, ut_ref, pi_ref, it_ref,
                    ucomb_ref, icomb_ref):
    i = pl.program_id(0)
    BLK = x_ref.shape[0]
    x = x_ref[...]
    valid = U_real - i * BLK
    rowmask = lax.broadcasted_iota(jnp.int32, (BLK, 1), 0) < valid
    pu = jnp.where(rowmask, pu_ref[...], 0.0).astype(jnp.bfloat16)

    ucomb_ref[...] = (
        jax.lax.dot_general(x, pi_ref[...], (((1,), (0,)), ((), ())),
                            preferred_element_type=jnp.float32)
        + ut_ref[...]
    )

    @pl.when(i == 0)
    def _():
        icomb_ref[...] = jnp.zeros_like(icomb_ref)

    icomb_ref[...] += jax.lax.dot_general(
        x, pu, (((0,), (0,)), ((), ())), preferred_element_type=jnp.float32)

    @pl.when(i == nsteps - 1)
    def _():
        icomb_ref[...] += it_ref[...]


def _tc_stream(x_pad, P_user, user_table, pi_pad, it_pad, U_real):
    UP, IP = x_pad.shape
    K = pi_pad.shape[1]
    nsteps = UP // _BLK
    return pl.pallas_call(
        functools.partial(_tc_stream_body, nsteps, U_real),
        grid=(nsteps,),
        in_specs=[
            pl.BlockSpec((_BLK, IP), lambda i: (i, 0)),
            pl.BlockSpec((_BLK, K), lambda i: (i, 0)),
            pl.BlockSpec((_BLK, K), lambda i: (i, 0)),
            pl.BlockSpec((IP, K), lambda i: (0, 0)),
            pl.BlockSpec((IP, K), lambda i: (0, 0)),
        ],
        out_specs=[
            pl.BlockSpec((_BLK, K), lambda i: (i, 0)),
            pl.BlockSpec((IP, K), lambda i: (0, 0)),
        ],
        out_shape=[
            jax.ShapeDtypeStruct((UP, K), jnp.float32),
            jax.ShapeDtypeStruct((IP, K), jnp.float32),
        ],
        compiler_params=pltpu.CompilerParams(
            dimension_semantics=("arbitrary",),
            fuse_transposed_lhs_in_matmul=True,
        ),
    )(x_pad, P_user, user_table, pi_pad, it_pad)


def _sc_gather_mul(user_idx, item_idx, U_comb, I_comb):
    B = user_idx.shape[0]
    K = U_comb.shape[1]
    info = plsc.get_sparse_core_info()
    NC, NS, L = info.num_cores, info.num_subcores, info.num_lanes
    NW = NC * NS
    assert B % NW == 0
    b_per_w = B // NW
    mesh = plsc.VectorSubcoreMesh(core_axis_name="c", subcore_axis_name="s")

    @functools.partial(
        pl.kernel,
        mesh=mesh,
        out_type=jax.ShapeDtypeStruct((B, K), jnp.float32),
        scratch_types=[
            pltpu.VMEM((b_per_w,), jnp.int32),
            pltpu.VMEM((b_per_w,), jnp.int32),
            pltpu.VMEM((b_per_w, K), jnp.float32),
            pltpu.VMEM((b_per_w, K), jnp.float32),
            pltpu.VMEM((b_per_w, K), jnp.float32),
            pltpu.SemaphoreType.DMA,
            pltpu.SemaphoreType.DMA,
        ],
        compiler_params=pltpu.CompilerParams(use_tc_tiling_on_sc=False),
    )
    def sc_k(uidx_hbm, iidx_hbm, ucomb_hbm, icomb_hbm, out_hbm,
             uidx_v, iidx_v, urows_v, irows_v, out_v, sem_u, sem_i):
        wid = lax.axis_index("s") * NC + lax.axis_index("c")
        base = wid * b_per_w
        pltpu.sync_copy(uidx_hbm.at[pl.ds(base, b_per_w)], uidx_v)
        pltpu.sync_copy(iidx_hbm.at[pl.ds(base, b_per_w)], iidx_v)
        cp_u = pltpu.async_copy(ucomb_hbm.at[uidx_v], urows_v, sem_u)
        cp_i = pltpu.async_copy(icomb_hbm.at[iidx_v], irows_v, sem_i)
        cp_u.wait()
        cp_i.wait()

        def body(r, carry):
            for h in range(K // L):
                sl = pl.ds(h * L, L)
                out_v[r, sl] = urows_v[r, sl] * irows_v[r, sl]
            return carry

        lax.fori_loop(0, b_per_w, body, 0)
        pltpu.sync_copy(out_v, out_hbm.at[pl.ds(base, b_per_w)])

    return sc_k(user_idx, item_idx, U_comb, I_comb)


def kernel(user_idx, item_idx, interactions, user_table, item_table,
           P_user, P_item):
    U, I = interactions.shape
    UP = ((U + _BLK - 1) // _BLK) * _BLK
    IP = 1024
    x_pad = jnp.pad(interactions.astype(jnp.bfloat16), ((0, UP - U), (0, IP - I)))
    pi_pad = jnp.pad(P_item, ((0, IP - I), (0, 0))).astype(jnp.bfloat16)
    it_pad = jnp.pad(item_table, ((0, IP - I), (0, 0)))
    U_comb, I_comb = _tc_stream(x_pad, P_user, user_table, pi_pad, it_pad, U)
    return _sc_gather_mul(user_idx.astype(jnp.int32),
                          item_idx.astype(jnp.int32), U_comb, I_comb)


# speedup vs baseline: 2.4238x; 2.4238x over previous
import functools

import jax
import jax.numpy as jnp
from jax import lax
from jax.experimental import pallas as pl
from jax.experimental.pallas import tpu as pltpu
from jax.experimental.pallas import tpu_sc as plsc

_CB = 2048


def _tc_stream_body(nsteps, U_real, x_ref, put_ref, utt_ref, pi_ref, it_ref,
                    ucomb_ref, icomb_ref):
    i = pl.program_id(0)
    CB = x_ref.shape[1]
    valid = U_real - i * CB
    colmask = lax.broadcasted_iota(jnp.int32, (1, CB), 1) < valid
    x = jnp.where(colmask, x_ref[...], 0.0).astype(jnp.bfloat16)
    put = jnp.where(colmask, put_ref[...], 0.0).astype(jnp.bfloat16)
    pi = pi_ref[...].astype(jnp.bfloat16)

    ucomb_ref[...] = (
        jax.lax.dot_general(x, pi, (((0,), (0,)), ((), ())),
                            preferred_element_type=jnp.float32)
        + utt_ref[...].T
    )

    @pl.when(i == 0)
    def _():
        icomb_ref[...] = jnp.zeros_like(icomb_ref)

    icomb_ref[...] += jax.lax.dot_general(
        x, put, (((1,), (1,)), ((), ())), preferred_element_type=jnp.float32)

    @pl.when(i == nsteps - 1)
    def _():
        icomb_ref[...] += it_ref[...]


def _tc_stream(xT, P_userT, user_tableT, P_item, item_table):
    I, U = xT.shape
    K = P_item.shape[1]
    nsteps = pl.cdiv(U, _CB)
    return pl.pallas_call(
        functools.partial(_tc_stream_body, nsteps, U),
        grid=(nsteps,),
        in_specs=[
            pl.BlockSpec((I, _CB), lambda i: (0, i)),
            pl.BlockSpec((K, _CB), lambda i: (0, i)),
            pl.BlockSpec((K, _CB), lambda i: (0, i)),
            pl.BlockSpec((I, K), lambda i: (0, 0)),
            pl.BlockSpec((I, K), lambda i: (0, 0)),
        ],
        out_specs=[
            pl.BlockSpec((_CB, K), lambda i: (i, 0)),
            pl.BlockSpec((I, K), lambda i: (0, 0)),
        ],
        out_shape=[
            jax.ShapeDtypeStruct((nsteps * _CB, K), jnp.float32),
            jax.ShapeDtypeStruct((I, K), jnp.float32),
        ],
        compiler_params=pltpu.CompilerParams(
            dimension_semantics=("arbitrary",),
            fuse_transposed_lhs_in_matmul=True,
        ),
    )(xT, P_userT, user_tableT, P_item, item_table)


def _sc_gather_mul(user_idx, item_idx, U_comb, I_comb):
    B = user_idx.shape[0]
    K = U_comb.shape[1]
    info = plsc.get_sparse_core_info()
    NC, NS, L = info.num_cores, info.num_subcores, info.num_lanes
    NW = NC * NS
    assert B % NW == 0
    b_per_w = B // NW
    mesh = plsc.VectorSubcoreMesh(core_axis_name="c", subcore_axis_name="s")

    @functools.partial(
        pl.kernel,
        mesh=mesh,
        out_type=jax.ShapeDtypeStruct((B, K), jnp.float32),
        scratch_types=[
            pltpu.VMEM((b_per_w,), jnp.int32),
            pltpu.VMEM((b_per_w,), jnp.int32),
            pltpu.VMEM((b_per_w, K), jnp.float32),
            pltpu.VMEM((b_per_w, K), jnp.float32),
            pltpu.VMEM((b_per_w, K), jnp.float32),
            pltpu.SemaphoreType.DMA,
            pltpu.SemaphoreType.DMA,
        ],
        compiler_params=pltpu.CompilerParams(use_tc_tiling_on_sc=False),
    )
    def sc_k(uidx_hbm, iidx_hbm, ucomb_hbm, icomb_hbm, out_hbm,
             uidx_v, iidx_v, urows_v, irows_v, out_v, sem_u, sem_i):
        wid = lax.axis_index("s") * NC + lax.axis_index("c")
        base = wid * b_per_w
        pltpu.sync_copy(uidx_hbm.at[pl.ds(base, b_per_w)], uidx_v)
        pltpu.sync_copy(iidx_hbm.at[pl.ds(base, b_per_w)], iidx_v)
        cp_u = pltpu.async_copy(ucomb_hbm.at[uidx_v], urows_v, sem_u)
        cp_i = pltpu.async_copy(icomb_hbm.at[iidx_v], irows_v, sem_i)
        cp_u.wait()
        cp_i.wait()

        def body(r, carry):
            for h in range(K // L):
                sl = pl.ds(h * L, L)
                out_v[r, sl] = urows_v[r, sl] * irows_v[r, sl]
            return carry

        lax.fori_loop(0, b_per_w, body, 0)
        pltpu.sync_copy(out_v, out_hbm.at[pl.ds(base, b_per_w)])

    return sc_k(user_idx, item_idx, U_comb, I_comb)


def kernel(user_idx, item_idx, interactions, user_table, item_table,
           P_user, P_item):
    U_comb, I_comb = _tc_stream(interactions.T, P_user.T, user_table.T,
                                P_item, item_table)
    return _sc_gather_mul(user_idx.astype(jnp.int32),
                          item_idx.astype(jnp.int32), U_comb, I_comb)


# speedup vs baseline: 4.0782x; 1.6825x over previous
import functools

import jax
import jax.numpy as jnp
from jax import lax
from jax.experimental import pallas as pl
from jax.experimental.pallas import tpu as pltpu
from jax.experimental.pallas import tpu_sc as plsc

_CB = 2048


def _tc_stream_body(nsteps, U_real, x_ref, put_ref, utt_ref, pi_ref, it_ref,
                    ucomb_ref, icomb_ref):
    i = pl.program_id(0)
    CB = x_ref.shape[1]
    valid = U_real - i * CB
    colmask = lax.broadcasted_iota(jnp.int32, (1, CB), 1) < valid
    x = x_ref[...].astype(jnp.bfloat16)
    put = jnp.where(colmask, put_ref[...], 0.0).astype(jnp.bfloat16)
    pi = pi_ref[...].astype(jnp.bfloat16)

    ucombT = jax.lax.dot_general(pi, x, (((0,), (0,)), ((), ())),
                                 preferred_element_type=jnp.float32)
    ucomb_ref[...] = (ucombT + utt_ref[...]).T

    @pl.when(i == 0)
    def _():
        icomb_ref[...] = jnp.zeros_like(icomb_ref)

    icomb_ref[...] += jax.lax.dot_general(
        x, put, (((1,), (1,)), ((), ())), preferred_element_type=jnp.float32)

    @pl.when(i == nsteps - 1)
    def _():
        icomb_ref[...] += it_ref[...]


def _tc_stream(xT, P_userT, user_tableT, P_item, item_table):
    I, U = xT.shape
    K = P_item.shape[1]
    nsteps = pl.cdiv(U, _CB)
    return pl.pallas_call(
        functools.partial(_tc_stream_body, nsteps, U),
        grid=(nsteps,),
        in_specs=[
            pl.BlockSpec((I, _CB), lambda i: (0, i)),
            pl.BlockSpec((K, _CB), lambda i: (0, i)),
            pl.BlockSpec((K, _CB), lambda i: (0, i)),
            pl.BlockSpec((I, K), lambda i: (0, 0)),
            pl.BlockSpec((I, K), lambda i: (0, 0)),
        ],
        out_specs=[
            pl.BlockSpec((_CB, K), lambda i: (i, 0)),
            pl.BlockSpec((I, K), lambda i: (0, 0)),
        ],
        out_shape=[
            jax.ShapeDtypeStruct((nsteps * _CB, K), jnp.float32),
            jax.ShapeDtypeStruct((I, K), jnp.float32),
        ],
        compiler_params=pltpu.CompilerParams(
            dimension_semantics=("arbitrary",),
            fuse_transposed_lhs_in_matmul=True,
        ),
    )(xT, P_userT, user_tableT, P_item, item_table)


def _sc_gather_mul(user_idx, item_idx, U_comb, I_comb):
    B = user_idx.shape[0]
    K = U_comb.shape[1]
    info = plsc.get_sparse_core_info()
    NC, NS, L = info.num_cores, info.num_subcores, info.num_lanes
    NW = NC * NS
    assert B % NW == 0
    b_per_w = B // NW
    mesh = plsc.VectorSubcoreMesh(core_axis_name="c", subcore_axis_name="s")

    @functools.partial(
        pl.kernel,
        mesh=mesh,
        out_type=jax.ShapeDtypeStruct((B, K), jnp.float32),
        scratch_types=[
            pltpu.VMEM((b_per_w,), jnp.int32),
            pltpu.VMEM((b_per_w,), jnp.int32),
            pltpu.VMEM((b_per_w, K), jnp.float32),
            pltpu.VMEM((b_per_w, K), jnp.float32),
            pltpu.VMEM((b_per_w, K), jnp.float32),
            pltpu.SemaphoreType.DMA,
            pltpu.SemaphoreType.DMA,
        ],
        compiler_params=pltpu.CompilerParams(use_tc_tiling_on_sc=False),
    )
    def sc_k(uidx_hbm, iidx_hbm, ucomb_hbm, icomb_hbm, out_hbm,
             uidx_v, iidx_v, urows_v, irows_v, out_v, sem_u, sem_i):
        wid = lax.axis_index("s") * NC + lax.axis_index("c")
        base = wid * b_per_w
        pltpu.sync_copy(uidx_hbm.at[pl.ds(base, b_per_w)], uidx_v)
        pltpu.sync_copy(iidx_hbm.at[pl.ds(base, b_per_w)], iidx_v)
        cp_u = pltpu.async_copy(ucomb_hbm.at[uidx_v], urows_v, sem_u)
        cp_i = pltpu.async_copy(icomb_hbm.at[iidx_v], irows_v, sem_i)
        cp_u.wait()
        cp_i.wait()

        def body(r, carry):
            for h in range(K // L):
                sl = pl.ds(h * L, L)
                out_v[r, sl] = urows_v[r, sl] * irows_v[r, sl]
            return carry

        lax.fori_loop(0, b_per_w, body, 0)
        pltpu.sync_copy(out_v, out_hbm.at[pl.ds(base, b_per_w)])

    return sc_k(user_idx, item_idx, U_comb, I_comb)


def kernel(user_idx, item_idx, interactions, user_table, item_table,
           P_user, P_item):
    U_comb, I_comb = _tc_stream(interactions.T, P_user.T, user_table.T,
                                P_item, item_table)
    return _sc_gather_mul(user_idx.astype(jnp.int32),
                          item_idx.astype(jnp.int32), U_comb, I_comb)


# speedup vs baseline: 4.7833x; 1.1729x over previous
import functools

import jax
import jax.numpy as jnp
from jax import lax
from jax.experimental import pallas as pl
from jax.experimental.pallas import tpu as pltpu
from jax.experimental.pallas import tpu_sc as plsc

_CB = 2048
_KP = 128


def _tc_stream_body(nsteps, U_real, K, x_ref, put_ref, utt_ref, pi_ref,
                    it_ref, ucomb_ref, icomb_ref):
    i = pl.program_id(0)
    CB = x_ref.shape[1]
    valid = U_real - i * CB
    colmask = lax.broadcasted_iota(jnp.int32, (1, CB), 1) < valid
    x = x_ref[...].astype(jnp.bfloat16)
    put = jnp.where(colmask, put_ref[...], 0.0).astype(jnp.bfloat16)
    pi = pi_ref[...].astype(jnp.bfloat16)

    ucombT = jax.lax.dot_general(pi, x, (((0,), (0,)), ((), ())),
                                 preferred_element_type=jnp.float32)
    ucomb_ref[:, 0:K] = (ucombT + utt_ref[...]).T

    @pl.when(i == 0)
    def _():
        icomb_ref[...] = jnp.zeros_like(icomb_ref)

    icomb_ref[:, 0:K] += jax.lax.dot_general(
        x, put, (((1,), (1,)), ((), ())), preferred_element_type=jnp.float32)

    @pl.when(i == nsteps - 1)
    def _():
        icomb_ref[:, 0:K] += it_ref[...]


def _tc_stream(xT, P_userT, user_tableT, P_item, item_table):
    I, U = xT.shape
    K = P_item.shape[1]
    nsteps = pl.cdiv(U, _CB)
    return pl.pallas_call(
        functools.partial(_tc_stream_body, nsteps, U, K),
        grid=(nsteps,),
        in_specs=[
            pl.BlockSpec((I, _CB), lambda i: (0, i)),
            pl.BlockSpec((K, _CB), lambda i: (0, i)),
            pl.BlockSpec((K, _CB), lambda i: (0, i)),
            pl.BlockSpec((I, K), lambda i: (0, 0)),
            pl.BlockSpec((I, K), lambda i: (0, 0)),
        ],
        out_specs=[
            pl.BlockSpec((_CB, _KP), lambda i: (i, 0)),
            pl.BlockSpec((I, _KP), lambda i: (0, 0)),
        ],
        out_shape=[
            jax.ShapeDtypeStruct((nsteps * _CB, _KP), jnp.float32),
            jax.ShapeDtypeStruct((I, _KP), jnp.float32),
        ],
        compiler_params=pltpu.CompilerParams(
            dimension_semantics=("arbitrary",),
            fuse_transposed_lhs_in_matmul=True,
        ),
    )(xT, P_userT, user_tableT, P_item, item_table)


def _sc_gather_mul(user_idx, item_idx, U_comb, I_comb, K):
    B = user_idx.shape[0]
    KP = U_comb.shape[1]
    info = plsc.get_sparse_core_info()
    NC, NS, L = info.num_cores, info.num_subcores, info.num_lanes
    NW = NC * NS
    assert B % NW == 0
    b_per_w = B // NW
    mesh = plsc.VectorSubcoreMesh(core_axis_name="c", subcore_axis_name="s")

    @functools.partial(
        pl.kernel,
        mesh=mesh,
        out_type=jax.ShapeDtypeStruct((B, K), jnp.float32),
        scratch_types=[
            pltpu.VMEM((b_per_w,), jnp.int32),
            pltpu.VMEM((b_per_w,), jnp.int32),
            pltpu.VMEM((b_per_w, KP), jnp.float32),
            pltpu.VMEM((b_per_w, KP), jnp.float32),
            pltpu.VMEM((b_per_w, K), jnp.float32),
            pltpu.SemaphoreType.DMA,
            pltpu.SemaphoreType.DMA,
        ],
        compiler_params=pltpu.CompilerParams(use_tc_tiling_on_sc=True),
    )
    def sc_k(uidx_hbm, iidx_hbm, ucomb_hbm, icomb_hbm, out_hbm,
             uidx_v, iidx_v, urows_v, irows_v, out_v, sem_u, sem_i):
        wid = lax.axis_index("s") * NC + lax.axis_index("c")
        base = wid * b_per_w
        pltpu.sync_copy(uidx_hbm.at[pl.ds(base, b_per_w)], uidx_v)
        pltpu.sync_copy(iidx_hbm.at[pl.ds(base, b_per_w)], iidx_v)
        cp_u = pltpu.async_copy(ucomb_hbm.at[uidx_v], urows_v, sem_u)
        cp_i = pltpu.async_copy(icomb_hbm.at[iidx_v], irows_v, sem_i)
        cp_u.wait()
        cp_i.wait()

        def body(r, carry):
            for h in range(K // L):
                sl = pl.ds(h * L, L)
                out_v[r, sl] = urows_v[r, sl] * irows_v[r, sl]
            return carry

        lax.fori_loop(0, b_per_w, body, 0)
        pltpu.sync_copy(out_v, out_hbm.at[pl.ds(base, b_per_w)])

    return sc_k(user_idx, item_idx, U_comb, I_comb)


def kernel(user_idx, item_idx, interactions, user_table, item_table,
           P_user, P_item):
    K = user_table.shape[1]
    U_comb, I_comb = _tc_stream(interactions.T, P_user.T, user_table.T,
                                P_item, item_table)
    return _sc_gather_mul(user_idx.astype(jnp.int32),
                          item_idx.astype(jnp.int32), U_comb, I_comb, K)


# speedup vs baseline: 5.1025x; 1.0667x over previous
import functools

import jax
import jax.numpy as jnp
from jax import lax
from jax.experimental import pallas as pl
from jax.experimental.pallas import tpu as pltpu
from jax.experimental.pallas import tpu_sc as plsc

_CB = 4096
_KP = 128


def _tc_stream_body(nsteps, U_real, K, x_ref, put_ref, utt_ref, pi_ref,
                    it_ref, ucomb_ref, icomb_ref):
    i = pl.program_id(0)
    CB = x_ref.shape[1]
    valid = U_real - i * CB
    colmask = lax.broadcasted_iota(jnp.int32, (1, CB), 1) < valid
    x = x_ref[...].astype(jnp.bfloat16)
    put = jnp.where(colmask, put_ref[...], 0.0).astype(jnp.bfloat16)
    pi = pi_ref[...].astype(jnp.bfloat16)

    ucombT = jax.lax.dot_general(pi, x, (((0,), (0,)), ((), ())),
                                 preferred_element_type=jnp.float32)
    ucomb_ref[:, 0:K] = (ucombT + utt_ref[...]).T

    @pl.when(i == 0)
    def _():
        icomb_ref[...] = jnp.zeros_like(icomb_ref)

    icomb_ref[:, 0:K] += jax.lax.dot_general(
        x, put, (((1,), (1,)), ((), ())), preferred_element_type=jnp.float32)

    @pl.when(i == nsteps - 1)
    def _():
        icomb_ref[:, 0:K] += it_ref[...]


def _tc_stream(xT, P_userT, user_tableT, P_item, item_table):
    I, U = xT.shape
    K = P_item.shape[1]
    nsteps = pl.cdiv(U, _CB)
    return pl.pallas_call(
        functools.partial(_tc_stream_body, nsteps, U, K),
        grid=(nsteps,),
        in_specs=[
            pl.BlockSpec((I, _CB), lambda i: (0, i)),
            pl.BlockSpec((K, _CB), lambda i: (0, i)),
            pl.BlockSpec((K, _CB), lambda i: (0, i)),
            pl.BlockSpec((I, K), lambda i: (0, 0)),
            pl.BlockSpec((I, K), lambda i: (0, 0)),
        ],
        out_specs=[
            pl.BlockSpec((_CB, _KP), lambda i: (i, 0)),
            pl.BlockSpec((I, _KP), lambda i: (0, 0)),
        ],
        out_shape=[
            jax.ShapeDtypeStruct((nsteps * _CB, _KP), jnp.float32),
            jax.ShapeDtypeStruct((I, _KP), jnp.float32),
        ],
        compiler_params=pltpu.CompilerParams(
            dimension_semantics=("arbitrary",),
            fuse_transposed_lhs_in_matmul=True,
        ),
    )(xT, P_userT, user_tableT, P_item, item_table)


def _sc_gather_mul(user_idx, item_idx, U_comb, I_comb, K):
    B = user_idx.shape[0]
    KP = U_comb.shape[1]
    info = plsc.get_sparse_core_info()
    NC, NS, L = info.num_cores, info.num_subcores, info.num_lanes
    NW = NC * NS
    assert B % NW == 0
    b_per_w = B // NW
    mesh = plsc.VectorSubcoreMesh(core_axis_name="c", subcore_axis_name="s")

    @functools.partial(
        pl.kernel,
        mesh=mesh,
        out_type=jax.ShapeDtypeStruct((B, K), jnp.float32),
        scratch_types=[
            pltpu.VMEM((b_per_w,), jnp.int32),
            pltpu.VMEM((b_per_w,), jnp.int32),
            pltpu.VMEM((b_per_w, KP), jnp.float32),
            pltpu.VMEM((b_per_w, KP), jnp.float32),
            pltpu.VMEM((b_per_w, K), jnp.float32),
            pltpu.SemaphoreType.DMA,
            pltpu.SemaphoreType.DMA,
        ],
        compiler_params=pltpu.CompilerParams(use_tc_tiling_on_sc=True),
    )
    def sc_k(uidx_hbm, iidx_hbm, ucomb_hbm, icomb_hbm, out_hbm,
             uidx_v, iidx_v, urows_v, irows_v, out_v, sem_u, sem_i):
        wid = lax.axis_index("s") * NC + lax.axis_index("c")
        base = wid * b_per_w
        pltpu.sync_copy(uidx_hbm.at[pl.ds(base, b_per_w)], uidx_v)
        pltpu.sync_copy(iidx_hbm.at[pl.ds(base, b_per_w)], iidx_v)
        cp_u = pltpu.async_copy(ucomb_hbm.at[uidx_v], urows_v, sem_u)
        cp_i = pltpu.async_copy(icomb_hbm.at[iidx_v], irows_v, sem_i)
        cp_u.wait()
        cp_i.wait()

        def body(r, carry):
            for h in range(K // L):
                sl = pl.ds(h * L, L)
                out_v[r, sl] = urows_v[r, sl] * irows_v[r, sl]
            return carry

        lax.fori_loop(0, b_per_w, body, 0)
        pltpu.sync_copy(out_v, out_hbm.at[pl.ds(base, b_per_w)])

    return sc_k(user_idx, item_idx, U_comb, I_comb)


def kernel(user_idx, item_idx, interactions, user_table, item_table,
           P_user, P_item):
    K = user_table.shape[1]
    U_comb, I_comb = _tc_stream(interactions.T, P_user.T, user_table.T,
                                P_item, item_table)
    return _sc_gather_mul(user_idx.astype(jnp.int32),
                          item_idx.astype(jnp.int32), U_comb, I_comb, K)
